# baseline (speedup 1.0000x reference)
import functools

import jax
import jax.numpy as jnp
from jax import lax
from jax.experimental import pallas as pl
from jax.experimental.pallas import tpu as pltpu
from jax.experimental.pallas import tpu_sc as plsc

DIM = 64
B = 16384
L2_REG = 1e-4

NC = 2
NS = 16
L = 16
NW = NC * NS
BPW = B // NW
GROUPS = BPW // L


def _sc_scores(user_table, item_table, users_id, pos_items_id, neg_items_id):
    mesh = plsc.VectorSubcoreMesh(core_axis_name="c", subcore_axis_name="s")

    @functools.partial(
        pl.kernel,
        mesh=mesh,
        compiler_params=pltpu.CompilerParams(
            needs_layout_passes=False, use_tc_tiling_on_sc=False
        ),
        out_type=(
            jax.ShapeDtypeStruct((B,), jnp.float32),
            jax.ShapeDtypeStruct((B,), jnp.float32),
            jax.ShapeDtypeStruct((NW, L), jnp.float32),
        ),
        scratch_types=[
            pltpu.VMEM((BPW,), jnp.int32),
            pltpu.VMEM((BPW,), jnp.int32),
            pltpu.VMEM((BPW,), jnp.int32),
            pltpu.VMEM((BPW, DIM), jnp.float32),
            pltpu.VMEM((BPW, DIM), jnp.float32),
            pltpu.VMEM((BPW, DIM), jnp.float32),
            pltpu.VMEM((BPW,), jnp.float32),
            pltpu.VMEM((BPW,), jnp.float32),
            pltpu.VMEM((L,), jnp.float32),
            pltpu.VMEM((L * L,), jnp.float32),
            pltpu.VMEM((L * L,), jnp.float32),
            pltpu.SemaphoreType.DMA,
            pltpu.SemaphoreType.DMA,
            pltpu.SemaphoreType.DMA,
        ],
    )
    def k(uid_hbm, pid_hbm, nid_hbm, ut_hbm, it_hbm,
          pos_out, neg_out, sq_out,
          uidx, pidx, nidx, urows, prows, nrows, psc, nsc, sqv, tpm, tnm,
          sem_u, sem_p, sem_n):
        wid = lax.axis_index("s") * NC + lax.axis_index("c")
        base = wid * BPW
        pltpu.sync_copy(uid_hbm.at[pl.ds(base, BPW)], uidx)
        pltpu.sync_copy(pid_hbm.at[pl.ds(base, BPW)], pidx)
        pltpu.sync_copy(nid_hbm.at[pl.ds(base, BPW)], nidx)
        cu = pltpu.async_copy(ut_hbm.at[uidx], urows, sem_u)
        cp = pltpu.async_copy(it_hbm.at[pidx], prows, sem_p)
        cn = pltpu.async_copy(it_hbm.at[nidx], nrows, sem_n)
        cu.wait()
        cp.wait()
        cn.wait()

        lanes = lax.iota(jnp.int32, L)

        def group(g, sq):
            gbase = pl.multiple_of(g * L, L)
            for r in range(L):
                tp = jnp.zeros((L,), jnp.float32)
                tn = jnp.zeros((L,), jnp.float32)
                for kk in range(DIM // L):
                    u = urows[gbase + r, pl.ds(kk * L, L)]
                    p = prows[gbase + r, pl.ds(kk * L, L)]
                    n = nrows[gbase + r, pl.ds(kk * L, L)]
                    tp = tp + u * p
                    tn = tn + u * n
                    sq = sq + u * u + p * p + n * n
                colidx = lanes * L + r
                plsc.store_scatter(tpm, [colidx], tp)
                plsc.store_scatter(tnm, [colidx], tn)
            pos_v = jnp.zeros((L,), jnp.float32)
            neg_v = jnp.zeros((L,), jnp.float32)
            for l in range(L):
                pos_v = pos_v + tpm[pl.ds(l * L, L)]
                neg_v = neg_v + tnm[pl.ds(l * L, L)]
            psc[pl.ds(gbase, L)] = pos_v
            nsc[pl.ds(gbase, L)] = neg_v
            return sq

        sq = lax.fori_loop(0, GROUPS, group, jnp.zeros((L,), jnp.float32))
        sqv[...] = sq
        pltpu.sync_copy(psc, pos_out.at[pl.ds(base, BPW)])
        pltpu.sync_copy(nsc, neg_out.at[pl.ds(base, BPW)])
        pltpu.sync_copy(sqv, sq_out.at[wid])

    return k(users_id, pos_items_id, neg_items_id, user_table, item_table)


def _tc_finalize(pos2, neg2, sq2):
    def body(p_ref, n_ref, s_ref, bpr_ref, auc_ref, reg_ref):
        p = p_ref[...]
        n = n_ref[...]
        d = n - p
        sp = jnp.maximum(d, 0.0) + jnp.log(1.0 + jnp.exp(-jnp.abs(d)))
        bpr_ref[0, 0] = jnp.sum(sp) * (1.0 / B)
        auc_ref[0, 0] = jnp.sum((p > n).astype(jnp.float32)) * (1.0 / B)
        reg_ref[0, 0] = (0.5 * L2_REG / B) * jnp.sum(s_ref[...])

    return pl.pallas_call(
        body,
        out_shape=(
            jax.ShapeDtypeStruct((1, 1), jnp.float32),
            jax.ShapeDtypeStruct((1, 1), jnp.float32),
            jax.ShapeDtypeStruct((1, 1), jnp.float32),
        ),
        out_specs=(
            pl.BlockSpec(memory_space=pltpu.SMEM),
            pl.BlockSpec(memory_space=pltpu.SMEM),
            pl.BlockSpec(memory_space=pltpu.SMEM),
        ),
    )(pos2, neg2, sq2)


def kernel(user_table, item_table, users_id, pos_items_id, neg_items_id):
    uid = users_id.astype(jnp.int32)
    pid = pos_items_id.astype(jnp.int32)
    nid = neg_items_id.astype(jnp.int32)
    pos_s, neg_s, sq = _sc_scores(user_table, item_table, uid, pid, nid)
    bpr, auc, reg = _tc_finalize(
        pos_s.reshape(128, 128), neg_s.reshape(128, 128), sq.reshape(4, 128)
    )
    return (bpr[0, 0], auc[0, 0], reg[0, 0])
